# baseline (speedup 1.0000x reference)
import functools

import jax
import jax.numpy as jnp
from jax.experimental import pallas as pl
from jax.experimental.pallas import tpu as pltpu


def _graph_kernel(D, H, L, RB,
                  nf_ref, adj_ref, ew_ref,
                  node_w_ref, node_b_ref,
                  wvf_ref, bvf_ref, wga_ref,
                  w1ab_ref, b1_ref, w2bc_ref, b2_ref,
                  logits_ref, loss_ref,
                  a_scr, bt_scr):
    N = adj_ref.shape[0]

    nf = nf_ref[...]
    h = (nf[:, 0:1] * node_w_ref[0:1, :]
         + nf[:, 1:2] * node_w_ref[1:2, :]
         + nf[:, 2:3] * node_w_ref[2:3, :]
         + node_b_ref[...])

    adj = adj_ref[...]
    for l in range(L):
        vp = jnp.dot(h, wvf_ref[l], preferred_element_type=jnp.float32) + bvf_ref[l]
        agg = jnp.dot(adj, vp[:, :D], preferred_element_type=jnp.float32)
        gate = jax.nn.sigmoid(
            vp[:, D:] + jnp.dot(agg, wga_ref[l], preferred_element_type=jnp.float32))
        h = jnp.maximum(h + gate * agg, 0.0)

    ab = jnp.dot(h, w1ab_ref[...], preferred_element_type=jnp.float32)
    a_scr[...] = ab[:, :H] + b1_ref[...]
    bt_scr[...] = jnp.transpose(ab[:, H:])

    b2 = b2_ref[0]

    def blk(i, acc):
        r0 = pl.multiple_of(i * RB, RB)
        at_blk = jnp.transpose(a_scr[pl.ds(r0, RB), :])
        for s in range(RB):
            ts = []
            for r in range(H // 8):
                ac = at_blk[8 * r:8 * r + 8, s:s + 1]
                ts.append(jnp.maximum(ac + bt_scr[8 * r:8 * r + 8, :], 0.0)
                          * w2bc_ref[8 * r:8 * r + 8, :])
            while len(ts) > 1:
                ts = [ts[j] + ts[j + 1] for j in range(0, len(ts), 2)]
            row = jnp.sum(ts[0], axis=0, keepdims=True) + b2
            logits_ref[pl.ds(r0 + s, 1), :] = row
        lg = logits_ref[pl.ds(r0, RB), :]
        d = lg * adj_ref[pl.ds(r0, RB), :] - ew_ref[pl.ds(r0, RB), :]
        return acc + jnp.sum(d * d)

    sq = jax.lax.fori_loop(0, N // RB, blk, jnp.zeros((1, 1), jnp.float32))
    loss_ref[...] = sq * (1.0 / float(N * N))


def kernel(node_w, node_b, wvf, bvf, wga, w1ab, b1, w2, b2,
           node_features, adj_matrix, edge_weights):
    B, N, _ = node_features.shape
    D = node_w.shape[1]
    L = wvf.shape[0]
    H = b1.shape[1]
    RB = 8
    ew = edge_weights[..., 0]
    w2bc = jnp.broadcast_to(jnp.reshape(w2, (H, 1)), (H, N))

    body = functools.partial(_graph_kernel, D, H, L, RB)

    def per_graph(shape):
        nd = len(shape)
        return pl.BlockSpec((None,) + shape, lambda b, _nd=nd: (b,) + (0,) * _nd)

    def resident(shape):
        nd = len(shape)
        return pl.BlockSpec(shape, lambda b, _nd=nd: (0,) * _nd)

    logits, loss = pl.pallas_call(
        body,
        out_shape=(jax.ShapeDtypeStruct((B, N, N), jnp.float32),
                   jax.ShapeDtypeStruct((B, 1, 1), jnp.float32)),
        grid=(B,),
        in_specs=[
            per_graph((N, 3)),
            per_graph((N, N)),
            per_graph((N, N)),
            resident((3, D)), resident((1, D)),
            resident((L, D, 2 * D)), resident((L, 1, 2 * D)),
            resident((L, D, D)),
            resident((D, 2 * H)), resident((1, H)),
            resident((H, N)),
            pl.BlockSpec(memory_space=pltpu.MemorySpace.SMEM),
        ],
        out_specs=(per_graph((N, N)), per_graph((1, 1))),
        scratch_shapes=[pltpu.VMEM((N, H), jnp.float32),
                        pltpu.VMEM((H, N), jnp.float32)],
        compiler_params=pltpu.CompilerParams(
            dimension_semantics=("parallel",)),
    )(node_features, adj_matrix, ew,
      node_w, node_b, wvf, bvf, wga, w1ab, b1, w2bc, b2)
    return logits, loss.reshape(B)


# speedup vs baseline: 1.0213x; 1.0213x over previous
import functools

import jax
import jax.numpy as jnp
from jax.experimental import pallas as pl
from jax.experimental.pallas import tpu as pltpu


def _graph_kernel(D, H, L, RB,
                  nf_ref, adj_ref, ew_ref,
                  node_w_ref, node_b_ref,
                  wvf_ref, bvf_ref, wga_ref,
                  w1ab_ref, b1_ref, w2bc_ref, b2_ref,
                  logits_ref, loss_ref,
                  a_scr, bt_scr):
    N = adj_ref.shape[0]

    nf = nf_ref[...]
    h = (nf[:, 0:1] * node_w_ref[0:1, :]
         + nf[:, 1:2] * node_w_ref[1:2, :]
         + nf[:, 2:3] * node_w_ref[2:3, :]
         + node_b_ref[...])

    adj = adj_ref[...]
    for l in range(L):
        vp = jnp.dot(h, wvf_ref[l], preferred_element_type=jnp.float32) + bvf_ref[l]
        agg = jnp.dot(adj, vp[:, :D], preferred_element_type=jnp.float32)
        gate = jax.nn.sigmoid(
            vp[:, D:] + jnp.dot(agg, wga_ref[l], preferred_element_type=jnp.float32))
        h = jnp.maximum(h + gate * agg, 0.0)

    ab = jnp.dot(h, w1ab_ref[...], preferred_element_type=jnp.float32)
    a_scr[...] = ab[:, :H] + b1_ref[...]
    bt_scr[...] = jnp.transpose(ab[:, H:])

    b2 = b2_ref[0]

    def blk(i, acc):
        r0 = pl.multiple_of(i * RB, RB)
        at_blk = jnp.transpose(a_scr[pl.ds(r0, RB), :])
        GR = 4
        for g in range(RB // GR):
            accs = [None] * GR
            for r in range(H // 8):
                btr = bt_scr[8 * r:8 * r + 8, :]
                w2r = w2bc_ref[8 * r:8 * r + 8, :]
                for s in range(GR):
                    ac = at_blk[8 * r:8 * r + 8, g * GR + s:g * GR + s + 1]
                    t = jnp.maximum(ac + btr, 0.0) * w2r
                    accs[s] = t if accs[s] is None else accs[s] + t
            for s in range(GR):
                row = jnp.sum(accs[s], axis=0, keepdims=True) + b2
                logits_ref[pl.ds(r0 + g * GR + s, 1), :] = row
        lg = logits_ref[pl.ds(r0, RB), :]
        d = lg * adj_ref[pl.ds(r0, RB), :] - ew_ref[pl.ds(r0, RB), :]
        return acc + jnp.sum(d * d)

    sq = jax.lax.fori_loop(0, N // RB, blk, jnp.zeros((1, 1), jnp.float32))
    loss_ref[...] = sq * (1.0 / float(N * N))


def kernel(node_w, node_b, wvf, bvf, wga, w1ab, b1, w2, b2,
           node_features, adj_matrix, edge_weights):
    B, N, _ = node_features.shape
    D = node_w.shape[1]
    L = wvf.shape[0]
    H = b1.shape[1]
    RB = 8
    ew = edge_weights[..., 0]
    w2bc = jnp.broadcast_to(jnp.reshape(w2, (H, 1)), (H, N))

    body = functools.partial(_graph_kernel, D, H, L, RB)

    def per_graph(shape):
        nd = len(shape)
        return pl.BlockSpec((None,) + shape, lambda b, _nd=nd: (b,) + (0,) * _nd)

    def resident(shape):
        nd = len(shape)
        return pl.BlockSpec(shape, lambda b, _nd=nd: (0,) * _nd)

    logits, loss = pl.pallas_call(
        body,
        out_shape=(jax.ShapeDtypeStruct((B, N, N), jnp.float32),
                   jax.ShapeDtypeStruct((B, 1, 1), jnp.float32)),
        grid=(B,),
        in_specs=[
            per_graph((N, 3)),
            per_graph((N, N)),
            per_graph((N, N)),
            resident((3, D)), resident((1, D)),
            resident((L, D, 2 * D)), resident((L, 1, 2 * D)),
            resident((L, D, D)),
            resident((D, 2 * H)), resident((1, H)),
            resident((H, N)),
            pl.BlockSpec(memory_space=pltpu.MemorySpace.SMEM),
        ],
        out_specs=(per_graph((N, N)), per_graph((1, 1))),
        scratch_shapes=[pltpu.VMEM((N, H), jnp.float32),
                        pltpu.VMEM((H, N), jnp.float32)],
        compiler_params=pltpu.CompilerParams(
            dimension_semantics=("parallel",)),
    )(node_features, adj_matrix, ew,
      node_w, node_b, wvf, bvf, wga, w1ab, b1, w2bc, b2)
    return logits, loss.reshape(B)


# speedup vs baseline: 1.4688x; 1.4381x over previous
import functools

import jax
import jax.numpy as jnp
from jax.experimental import pallas as pl
from jax.experimental.pallas import tpu as pltpu


def _graph_kernel(D, H, L, RB,
                  nf_ref, adj_ref, ew_ref,
                  node_w_ref, node_b_ref,
                  wvf_ref, bvf_ref, wga_ref,
                  w1ab_ref, b1_ref, w2bc_ref, b2_ref,
                  logits_ref, loss_ref,
                  a_scr, bt_scr):
    N = adj_ref.shape[0]

    nf = nf_ref[...]
    h = (nf[:, 0:1] * node_w_ref[0:1, :]
         + nf[:, 1:2] * node_w_ref[1:2, :]
         + nf[:, 2:3] * node_w_ref[2:3, :]
         + node_b_ref[...])

    adj = adj_ref[...]
    for l in range(L):
        vp = jnp.dot(h, wvf_ref[l], preferred_element_type=jnp.float32) + bvf_ref[l]
        agg = jnp.dot(adj, vp[:, :D], preferred_element_type=jnp.float32)
        gate = jax.nn.sigmoid(
            vp[:, D:] + jnp.dot(agg, wga_ref[l], preferred_element_type=jnp.float32))
        h = jnp.maximum(h + gate * agg, 0.0)

    ab = jnp.dot(h, w1ab_ref[...], preferred_element_type=jnp.float32)
    a_scr[...] = ab[:, :H] + b1_ref[...]
    bt_scr[...] = jnp.transpose(ab[:, H:])

    b2 = b2_ref[0]

    def blk(i, acc):
        r0 = pl.multiple_of(i * RB, RB)
        at_blk = jnp.transpose(a_scr[pl.ds(r0, RB), :])
        for s in range(RB):
            pacc = None
            for r in range(H // 8):
                ac = at_blk[8 * r:8 * r + 8, s:s + 1]
                t = (jnp.maximum(ac + bt_scr[8 * r:8 * r + 8, :], 0.0)
                     * w2bc_ref[8 * r:8 * r + 8, :])
                pacc = t if pacc is None else pacc + t
            row = jnp.sum(pacc, axis=0, keepdims=True) + b2
            logits_ref[pl.ds(r0 + s, 1), :] = row
        lg = logits_ref[pl.ds(r0, RB), :]
        d = lg * adj_ref[pl.ds(r0, RB), :] - ew_ref[pl.ds(r0, RB), :]
        d2 = d * d
        part = d2[0:8, :]
        for q in range(1, RB // 8):
            part = part + d2[8 * q:8 * q + 8, :]
        return acc + part

    sq_vec = jax.lax.fori_loop(0, N // RB, blk,
                               jnp.zeros((8, N), jnp.float32))
    loss_ref[...] = jnp.sum(sq_vec).reshape(1, 1) * (1.0 / float(N * N))


def kernel(node_w, node_b, wvf, bvf, wga, w1ab, b1, w2, b2,
           node_features, adj_matrix, edge_weights):
    B, N, _ = node_features.shape
    D = node_w.shape[1]
    L = wvf.shape[0]
    H = b1.shape[1]
    RB = 16
    ew = edge_weights[..., 0]
    w2bc = jnp.broadcast_to(jnp.reshape(w2, (H, 1)), (H, N))

    body = functools.partial(_graph_kernel, D, H, L, RB)

    def per_graph(shape):
        nd = len(shape)
        return pl.BlockSpec((None,) + shape, lambda b, _nd=nd: (b,) + (0,) * _nd)

    def resident(shape):
        nd = len(shape)
        return pl.BlockSpec(shape, lambda b, _nd=nd: (0,) * _nd)

    logits, loss = pl.pallas_call(
        body,
        out_shape=(jax.ShapeDtypeStruct((B, N, N), jnp.float32),
                   jax.ShapeDtypeStruct((B, 1, 1), jnp.float32)),
        grid=(B,),
        in_specs=[
            per_graph((N, 3)),
            per_graph((N, N)),
            per_graph((N, N)),
            resident((3, D)), resident((1, D)),
            resident((L, D, 2 * D)), resident((L, 1, 2 * D)),
            resident((L, D, D)),
            resident((D, 2 * H)), resident((1, H)),
            resident((H, N)),
            pl.BlockSpec(memory_space=pltpu.MemorySpace.SMEM),
        ],
        out_specs=(per_graph((N, N)), per_graph((1, 1))),
        scratch_shapes=[pltpu.VMEM((N, H), jnp.float32),
                        pltpu.VMEM((H, N), jnp.float32)],
        compiler_params=pltpu.CompilerParams(
            dimension_semantics=("parallel",)),
    )(node_features, adj_matrix, ew,
      node_w, node_b, wvf, bvf, wga, w1ab, b1, w2bc, b2)
    return logits, loss.reshape(B)


# speedup vs baseline: 1.6689x; 1.1362x over previous
import functools

import jax
import jax.numpy as jnp
from jax.experimental import pallas as pl
from jax.experimental.pallas import tpu as pltpu


def _graph_kernel(D, H, L, RB,
                  nf_ref, adj_ref, ew_ref,
                  node_w_ref, node_b_ref,
                  wvf_ref, bvf_ref, wga_ref,
                  w1ab_ref, b1_ref, w2bc_ref, b2_ref,
                  logits_ref, loss_ref,
                  a_scr, bt_scr):
    N = adj_ref.shape[0]

    nf = nf_ref[...]
    h = (nf[:, 0:1] * node_w_ref[0:1, :]
         + nf[:, 1:2] * node_w_ref[1:2, :]
         + nf[:, 2:3] * node_w_ref[2:3, :]
         + node_b_ref[...])

    adj = adj_ref[...]
    for l in range(L):
        vp = jnp.dot(h, wvf_ref[l], preferred_element_type=jnp.float32) + bvf_ref[l]
        agg = jnp.dot(adj, vp[:, :D], preferred_element_type=jnp.float32)
        gate = jax.nn.sigmoid(
            vp[:, D:] + jnp.dot(agg, wga_ref[l], preferred_element_type=jnp.float32))
        h = jnp.maximum(h + gate * agg, 0.0)

    ab = jnp.dot(h, w1ab_ref[...], preferred_element_type=jnp.float32)
    a_scr[...] = ab[:, :H] + b1_ref[...]
    bt_scr[...] = jnp.transpose(ab[:, H:])

    b2 = b2_ref[0]

    sub_iota = jax.lax.broadcasted_iota(jnp.int32, (8, N), 0)

    def blk(i, acc):
        r0 = pl.multiple_of(i * RB, RB)
        at_blk = jnp.transpose(a_scr[pl.ds(r0, RB), :])
        for g in range(RB // 8):
            lg8 = None
            for s8 in range(8):
                s = g * 8 + s8
                pacc = None
                for r in range(H // 8):
                    ac = at_blk[8 * r:8 * r + 8, s:s + 1]
                    t = (jnp.maximum(ac + bt_scr[8 * r:8 * r + 8, :], 0.0)
                         * w2bc_ref[8 * r:8 * r + 8, :])
                    pacc = t if pacc is None else pacc + t
                row = jnp.sum(pacc, axis=0, keepdims=True)
                rowb = jax.lax.broadcast_in_dim(row, (8, N), (0, 1))
                lg8 = rowb if lg8 is None else jnp.where(sub_iota == s8,
                                                         rowb, lg8)
            lg8 = lg8 + b2
            q0 = pl.multiple_of(r0 + g * 8, 8)
            logits_ref[pl.ds(q0, 8), :] = lg8
            d = (lg8 * adj_ref[pl.ds(q0, 8), :] - ew_ref[pl.ds(q0, 8), :])
            acc = acc + d * d
        return acc

    sq_vec = jax.lax.fori_loop(0, N // RB, blk,
                               jnp.zeros((8, N), jnp.float32))
    loss_ref[...] = jnp.sum(sq_vec).reshape(1, 1) * (1.0 / float(N * N))


def kernel(node_w, node_b, wvf, bvf, wga, w1ab, b1, w2, b2,
           node_features, adj_matrix, edge_weights):
    B, N, _ = node_features.shape
    D = node_w.shape[1]
    L = wvf.shape[0]
    H = b1.shape[1]
    RB = 32
    ew = edge_weights[..., 0]
    w2bc = jnp.broadcast_to(jnp.reshape(w2, (H, 1)), (H, N))

    body = functools.partial(_graph_kernel, D, H, L, RB)

    def per_graph(shape):
        nd = len(shape)
        return pl.BlockSpec((None,) + shape, lambda b, _nd=nd: (b,) + (0,) * _nd)

    def resident(shape):
        nd = len(shape)
        return pl.BlockSpec(shape, lambda b, _nd=nd: (0,) * _nd)

    logits, loss = pl.pallas_call(
        body,
        out_shape=(jax.ShapeDtypeStruct((B, N, N), jnp.float32),
                   jax.ShapeDtypeStruct((B, 1, 1), jnp.float32)),
        grid=(B,),
        in_specs=[
            per_graph((N, 3)),
            per_graph((N, N)),
            per_graph((N, N)),
            resident((3, D)), resident((1, D)),
            resident((L, D, 2 * D)), resident((L, 1, 2 * D)),
            resident((L, D, D)),
            resident((D, 2 * H)), resident((1, H)),
            resident((H, N)),
            pl.BlockSpec(memory_space=pltpu.MemorySpace.SMEM),
        ],
        out_specs=(per_graph((N, N)), per_graph((1, 1))),
        scratch_shapes=[pltpu.VMEM((N, H), jnp.float32),
                        pltpu.VMEM((H, N), jnp.float32)],
        compiler_params=pltpu.CompilerParams(
            dimension_semantics=("parallel",)),
    )(node_features, adj_matrix, ew,
      node_w, node_b, wvf, bvf, wga, w1ab, b1, w2bc, b2)
    return logits, loss.reshape(B)


# speedup vs baseline: 1.7536x; 1.0507x over previous
import functools

import jax
import jax.numpy as jnp
from jax.experimental import pallas as pl
from jax.experimental.pallas import tpu as pltpu


def _graph_kernel(D, H, L, RB,
                  nf_ref, adj_ref, ew_ref,
                  node_w_ref, node_b_ref,
                  wvf_ref, bvf_ref, wga_ref,
                  w1ab_ref, b1_ref, w2bc_ref, b2_ref,
                  logits_ref, loss_ref,
                  a_scr, bt_scr):
    N = adj_ref.shape[0]

    nf = nf_ref[...]
    h = (nf[:, 0:1] * node_w_ref[0:1, :]
         + nf[:, 1:2] * node_w_ref[1:2, :]
         + nf[:, 2:3] * node_w_ref[2:3, :]
         + node_b_ref[...])

    adj = adj_ref[...]
    for l in range(L):
        vp = jnp.dot(h, wvf_ref[l], preferred_element_type=jnp.float32) + bvf_ref[l]
        agg = jnp.dot(adj, vp[:, :D], preferred_element_type=jnp.float32)
        gate = jax.nn.sigmoid(
            vp[:, D:] + jnp.dot(agg, wga_ref[l], preferred_element_type=jnp.float32))
        h = jnp.maximum(h + gate * agg, 0.0)

    ab = jnp.dot(h, w1ab_ref[...], preferred_element_type=jnp.float32)
    a_scr[...] = ab[:, :H] + b1_ref[...]
    bt_scr[...] = jnp.transpose(ab[:, H:])

    b2 = b2_ref[0]

    sub_iota = jax.lax.broadcasted_iota(jnp.int32, (8, N), 0)

    def blk(i, acc):
        r0 = pl.multiple_of(i * RB, RB)
        at_blk = jnp.transpose(a_scr[pl.ds(r0, RB), :])
        for g in range(RB // 8):
            lg8 = None
            for s8 in range(8):
                s = g * 8 + s8
                pacc = None
                for r in range(H // 8):
                    ac = at_blk[8 * r:8 * r + 8, s:s + 1]
                    t = (jnp.maximum(ac + bt_scr[8 * r:8 * r + 8, :], 0.0)
                         * w2bc_ref[8 * r:8 * r + 8, :])
                    pacc = t if pacc is None else pacc + t
                row = jnp.sum(pacc, axis=0, keepdims=True)
                rowb = jax.lax.broadcast_in_dim(row, (8, N), (0, 1))
                lg8 = rowb if lg8 is None else jnp.where(sub_iota == s8,
                                                         rowb, lg8)
            lg8 = lg8 + b2
            q0 = pl.multiple_of(r0 + g * 8, 8)
            logits_ref[pl.ds(q0, 8), :] = lg8
            d = (lg8 * adj_ref[pl.ds(q0, 8), :] - ew_ref[pl.ds(q0, 8), :])
            acc = acc + d * d
        return acc

    sq_vec = jax.lax.fori_loop(0, N // RB, blk,
                               jnp.zeros((8, N), jnp.float32))
    loss_ref[...] = jnp.sum(sq_vec).reshape(1, 1) * (1.0 / float(N * N))


def kernel(node_w, node_b, wvf, bvf, wga, w1ab, b1, w2, b2,
           node_features, adj_matrix, edge_weights):
    B, N, _ = node_features.shape
    D = node_w.shape[1]
    L = wvf.shape[0]
    H = b1.shape[1]
    RB = 64
    ew = edge_weights[..., 0]
    w2bc = jnp.broadcast_to(jnp.reshape(w2, (H, 1)), (H, N))

    body = functools.partial(_graph_kernel, D, H, L, RB)

    def per_graph(shape):
        nd = len(shape)
        return pl.BlockSpec((None,) + shape, lambda b, _nd=nd: (b,) + (0,) * _nd)

    def resident(shape):
        nd = len(shape)
        return pl.BlockSpec(shape, lambda b, _nd=nd: (0,) * _nd)

    logits, loss = pl.pallas_call(
        body,
        out_shape=(jax.ShapeDtypeStruct((B, N, N), jnp.float32),
                   jax.ShapeDtypeStruct((B, 1, 1), jnp.float32)),
        grid=(B,),
        in_specs=[
            per_graph((N, 3)),
            per_graph((N, N)),
            per_graph((N, N)),
            resident((3, D)), resident((1, D)),
            resident((L, D, 2 * D)), resident((L, 1, 2 * D)),
            resident((L, D, D)),
            resident((D, 2 * H)), resident((1, H)),
            resident((H, N)),
            pl.BlockSpec(memory_space=pltpu.MemorySpace.SMEM),
        ],
        out_specs=(per_graph((N, N)), per_graph((1, 1))),
        scratch_shapes=[pltpu.VMEM((N, H), jnp.float32),
                        pltpu.VMEM((H, N), jnp.float32)],
        compiler_params=pltpu.CompilerParams(
            dimension_semantics=("parallel",)),
    )(node_features, adj_matrix, ew,
      node_w, node_b, wvf, bvf, wga, w1ab, b1, w2bc, b2)
    return logits, loss.reshape(B)


# speedup vs baseline: 1.8075x; 1.0307x over previous
import functools

import jax
import jax.numpy as jnp
from jax.experimental import pallas as pl
from jax.experimental.pallas import tpu as pltpu


def _graph_kernel(D, H, L, RB,
                  nf_ref, adj_ref, ew_ref,
                  node_w_ref, node_b_ref,
                  wvf_ref, bvf_ref, wga_ref,
                  w1ab_ref, b1_ref, w2bc_ref, b2_ref,
                  logits_ref, loss_ref,
                  a_scr, bt_scr):
    N = adj_ref.shape[0]

    nf = nf_ref[...]
    h = (nf[:, 0:1] * node_w_ref[0:1, :]
         + nf[:, 1:2] * node_w_ref[1:2, :]
         + nf[:, 2:3] * node_w_ref[2:3, :]
         + node_b_ref[...])

    adj = adj_ref[...]
    for l in range(L):
        vp = jnp.dot(h, wvf_ref[l], preferred_element_type=jnp.float32) + bvf_ref[l]
        agg = jnp.dot(adj, vp[:, :D], preferred_element_type=jnp.float32)
        gate = jax.nn.sigmoid(
            vp[:, D:] + jnp.dot(agg, wga_ref[l], preferred_element_type=jnp.float32))
        h = jnp.maximum(h + gate * agg, 0.0)

    ab = jnp.dot(h, w1ab_ref[...], preferred_element_type=jnp.float32)
    a_scr[...] = ab[:, :H] + b1_ref[...]
    bt_scr[...] = jnp.transpose(ab[:, H:])

    b2 = b2_ref[0]

    sub_iota = jax.lax.broadcasted_iota(jnp.int32, (8, N), 0)

    def blk(i, acc):
        r0 = pl.multiple_of(i * RB, RB)
        at_blk = jnp.transpose(a_scr[pl.ds(r0, RB), :])
        for g in range(RB // 8):
            lg8 = None
            for s8 in range(8):
                s = g * 8 + s8
                pacc = None
                for r in range(H // 8):
                    ac = at_blk[8 * r:8 * r + 8, s:s + 1]
                    t = (jnp.maximum(ac + bt_scr[8 * r:8 * r + 8, :], 0.0)
                         * w2bc_ref[8 * r:8 * r + 8, :])
                    pacc = t if pacc is None else pacc + t
                row = jnp.sum(pacc, axis=0, keepdims=True)
                rowb = jax.lax.broadcast_in_dim(row, (8, N), (0, 1))
                lg8 = rowb if lg8 is None else jnp.where(sub_iota == s8,
                                                         rowb, lg8)
            lg8 = lg8 + b2
            q0 = pl.multiple_of(r0 + g * 8, 8)
            logits_ref[pl.ds(q0, 8), :] = lg8
            d = (lg8 * adj_ref[pl.ds(q0, 8), :] - ew_ref[pl.ds(q0, 8), :])
            acc = acc + d * d
        return acc

    sq_vec = jax.lax.fori_loop(0, N // RB, blk,
                               jnp.zeros((8, N), jnp.float32))
    loss_ref[...] = jnp.sum(sq_vec).reshape(1, 1) * (1.0 / float(N * N))


def kernel(node_w, node_b, wvf, bvf, wga, w1ab, b1, w2, b2,
           node_features, adj_matrix, edge_weights):
    B, N, _ = node_features.shape
    D = node_w.shape[1]
    L = wvf.shape[0]
    H = b1.shape[1]
    RB = 128
    ew = edge_weights[..., 0]
    w2bc = jnp.broadcast_to(jnp.reshape(w2, (H, 1)), (H, N))

    body = functools.partial(_graph_kernel, D, H, L, RB)

    def per_graph(shape):
        nd = len(shape)
        return pl.BlockSpec((None,) + shape, lambda b, _nd=nd: (b,) + (0,) * _nd)

    def resident(shape):
        nd = len(shape)
        return pl.BlockSpec(shape, lambda b, _nd=nd: (0,) * _nd)

    logits, loss = pl.pallas_call(
        body,
        out_shape=(jax.ShapeDtypeStruct((B, N, N), jnp.float32),
                   jax.ShapeDtypeStruct((B, 1, 1), jnp.float32)),
        grid=(B,),
        in_specs=[
            per_graph((N, 3)),
            per_graph((N, N)),
            per_graph((N, N)),
            resident((3, D)), resident((1, D)),
            resident((L, D, 2 * D)), resident((L, 1, 2 * D)),
            resident((L, D, D)),
            resident((D, 2 * H)), resident((1, H)),
            resident((H, N)),
            pl.BlockSpec(memory_space=pltpu.MemorySpace.SMEM),
        ],
        out_specs=(per_graph((N, N)), per_graph((1, 1))),
        scratch_shapes=[pltpu.VMEM((N, H), jnp.float32),
                        pltpu.VMEM((H, N), jnp.float32)],
        compiler_params=pltpu.CompilerParams(
            dimension_semantics=("parallel",)),
    )(node_features, adj_matrix, ew,
      node_w, node_b, wvf, bvf, wga, w1ab, b1, w2bc, b2)
    return logits, loss.reshape(B)
